# Initial kernel scaffold; baseline (speedup 1.0000x reference)
import functools

import jax
import jax.numpy as jnp
from jax import lax
from jax.experimental import pallas as pl
from jax.experimental.pallas import tpu as pltpu
from jax.experimental.pallas import tpu_sc as plsc

_N = 10000
_E = 320000
_H = 128
_G = 128
_C = 10

_NC = 2
_NS = 16
_EPC = _E // _NC
_EPW = _E // (_NC * _NS)
_CHUNK = 128
_NFULL = _EPW // _CHUNK
_TAIL = _EPW - _NFULL * _CHUNK
_RPW = _N // _NS
_ZREP = 125

_BN = 1000


def _sc_aggregate(xin, src, dst):
    mesh = plsc.VectorSubcoreMesh(core_axis_name="c", subcore_axis_name="s")

    @functools.partial(
        pl.kernel,
        out_type=jax.ShapeDtypeStruct((_NC * _N, _H), jnp.float32),
        mesh=mesh,
        scratch_types=[
            pltpu.VMEM((_CHUNK,), jnp.int32),
            pltpu.VMEM((_CHUNK,), jnp.int32),
            pltpu.VMEM((_TAIL,), jnp.int32),
            pltpu.VMEM((_TAIL,), jnp.int32),
            pltpu.VMEM((_CHUNK, _H), jnp.float32),
            pltpu.VMEM_SHARED((_N, _H), jnp.float32),
            pltpu.SemaphoreType.DMA,
        ],
    )
    def agg(x_hbm, src_hbm, dst_hbm, out_hbm,
            src_v, dst_v, srct_v, dstt_v, rows_v, acc_sh, sem):
        c = lax.axis_index("c")
        s = lax.axis_index("s")

        def zero_body(i, _):
            for j in range(_H // 16):
                rows_v[i, pl.ds(j * 16, 16)] = jnp.zeros((16,), jnp.float32)
            return 0
        lax.fori_loop(0, _ZREP, zero_body, 0)
        r0 = s * _RPW
        for k in range(_RPW // _ZREP):
            pltpu.sync_copy(rows_v.at[pl.ds(0, _ZREP)],
                            acc_sh.at[pl.ds(r0 + k * _ZREP, _ZREP)])
        plsc.subcore_barrier()

        ebase = c * _EPC + s * _EPW

        def edge_body(t, _):
            e0 = ebase + t * _CHUNK
            pltpu.sync_copy(src_hbm.at[pl.ds(e0, _CHUNK)], src_v)
            pltpu.sync_copy(dst_hbm.at[pl.ds(e0, _CHUNK)], dst_v)
            pltpu.async_copy(x_hbm.at[src_v], rows_v, sem).wait()
            pltpu.sync_copy(rows_v, acc_sh.at[dst_v], add=True)
            return 0
        lax.fori_loop(0, _NFULL, edge_body, 0)

        et = ebase + _NFULL * _CHUNK
        pltpu.sync_copy(src_hbm.at[pl.ds(et, _TAIL)], srct_v)
        pltpu.sync_copy(dst_hbm.at[pl.ds(et, _TAIL)], dstt_v)
        pltpu.async_copy(x_hbm.at[srct_v], rows_v.at[pl.ds(0, _TAIL)], sem).wait()
        pltpu.sync_copy(rows_v.at[pl.ds(0, _TAIL)], acc_sh.at[dstt_v], add=True)

        plsc.subcore_barrier()
        pltpu.sync_copy(acc_sh.at[pl.ds(r0, _RPW)],
                        out_hbm.at[pl.ds(c * _N + r0, _RPW)])

    return agg(xin, src, dst)


def _gin_dense(z, agg0, agg1, wm, bias_row, eps_ref):
    return jnp.maximum(
        lax.dot_general((1.0 + eps_ref[0, 0]) * z + agg0 + agg1, wm,
                        (((1,), (1,)), ((), ())),
                        preferred_element_type=jnp.float32) + bias_row, 0.0)


def _tc_layer1(xin, parts, W, mask, b, eps):
    def body(eps_ref, x_ref, p0_ref, p1_ref, w_ref, m_ref, b_ref, o_ref):
        wm = w_ref[...] * m_ref[...]
        o_ref[...] = _gin_dense(x_ref[...], p0_ref[...], p1_ref[...], wm,
                                b_ref[...], eps_ref)

    nb = _N // _BN
    return pl.pallas_call(
        body,
        grid=(nb,),
        in_specs=[
            pl.BlockSpec(memory_space=pltpu.SMEM),
            pl.BlockSpec((_BN, _H), lambda i: (i, 0)),
            pl.BlockSpec((_BN, _H), lambda i: (i, 0)),
            pl.BlockSpec((_BN, _H), lambda i: (i + nb, 0)),
            pl.BlockSpec((_H, _H), lambda i: (0, 0)),
            pl.BlockSpec((_H, _H), lambda i: (0, 0)),
            pl.BlockSpec((1, _H), lambda i: (0, 0)),
        ],
        out_specs=pl.BlockSpec((_BN, _H), lambda i: (i, 0)),
        out_shape=jax.ShapeDtypeStruct((_N, _H), jnp.float32),
    )(eps.reshape(1, 1), xin, parts, parts, W, mask, b.reshape(1, _H))


def _tc_layer2_pool(zin, parts, W, mask, b, eps, batch_col, Wp_pad, bp_pad):
    nb = _N // _BN

    def body(eps_ref, z_ref, p0_ref, p1_ref, w_ref, m_ref, b_ref,
             bt_ref, wp_ref, bp_ref, o_ref, pool_acc, cnt_acc):
        i = pl.program_id(0)
        wm = w_ref[...] * m_ref[...]
        z2 = _gin_dense(z_ref[...], p0_ref[...], p1_ref[...], wm,
                        b_ref[...], eps_ref)
        gid = lax.broadcasted_iota(jnp.int32, (_BN, _G), 1)
        onehot = (bt_ref[...] == gid).astype(jnp.float32)

        @pl.when(i == 0)
        def _():
            pool_acc[...] = jnp.zeros_like(pool_acc)
            cnt_acc[...] = jnp.zeros_like(cnt_acc)

        dn = (((0,), (0,)), ((), ()))
        pool_acc[...] += lax.dot_general(onehot, z2, dn,
                                         preferred_element_type=jnp.float32)
        cnt_acc[...] += lax.dot_general(onehot, jnp.ones((_BN, _H), jnp.float32),
                                        dn, preferred_element_type=jnp.float32)

        @pl.when(i == nb - 1)
        def _():
            mean = pool_acc[...] / jnp.maximum(cnt_acc[...], 1.0)
            y = lax.dot_general(mean, wp_ref[...], (((1,), (1,)), ((), ())),
                                preferred_element_type=jnp.float32) + bp_ref[...]
            o_ref[...] = jax.nn.sigmoid(y)

    return pl.pallas_call(
        body,
        grid=(nb,),
        in_specs=[
            pl.BlockSpec(memory_space=pltpu.SMEM),
            pl.BlockSpec((_BN, _H), lambda i: (i, 0)),
            pl.BlockSpec((_BN, _H), lambda i: (i, 0)),
            pl.BlockSpec((_BN, _H), lambda i: (i + nb, 0)),
            pl.BlockSpec((_H, _H), lambda i: (0, 0)),
            pl.BlockSpec((_H, _H), lambda i: (0, 0)),
            pl.BlockSpec((1, _H), lambda i: (0, 0)),
            pl.BlockSpec((_BN, 1), lambda i: (i, 0)),
            pl.BlockSpec((_H, _H), lambda i: (0, 0)),
            pl.BlockSpec((1, _H), lambda i: (0, 0)),
        ],
        out_specs=pl.BlockSpec((_G, _H), lambda i: (0, 0)),
        out_shape=jax.ShapeDtypeStruct((_G, _H), jnp.float32),
        scratch_shapes=[
            pltpu.VMEM((_G, _H), jnp.float32),
            pltpu.VMEM((_G, _H), jnp.float32),
        ],
    )(eps.reshape(1, 1), zin, parts, parts, W, mask, b.reshape(1, _H),
      batch_col, Wp_pad, bp_pad)


def kernel(x, edge_index, batch, weights, W1, b1, eps1, W2, b2, eps2, Wp, bp):
    src = edge_index[0]
    dst = edge_index[1]
    batch_col = batch.reshape(_N, 1)
    Wp_pad = jnp.zeros((_H, _H), jnp.float32).at[:_C, :].set(Wp)
    bp_pad = jnp.zeros((1, _H), jnp.float32).at[0, :_C].set(bp)

    parts1 = _sc_aggregate(x, src, dst)
    z1 = _tc_layer1(x, parts1, W1, weights, b1, eps1)
    parts2 = _sc_aggregate(z1, src, dst)
    full = _tc_layer2_pool(z1, parts2, W2, weights, b2, eps2,
                           batch_col, Wp_pad, bp_pad)
    return full[:, :_C]


# speedup vs baseline: 6.5656x; 6.5656x over previous
import functools

import jax
import jax.numpy as jnp
from jax import lax
from jax.experimental import pallas as pl
from jax.experimental.pallas import tpu as pltpu
from jax.experimental.pallas import tpu_sc as plsc

_N = 10000
_E = 320000
_H = 128
_G = 128
_C = 10

_NC = 2
_NS = 16
_EPC = _E // _NC
_EPW = _E // (_NC * _NS)
_CHUNK = 128
_NFULL = _EPW // _CHUNK
_TAIL = _EPW - _NFULL * _CHUNK
_RPW = 624
_RTAIL = _N - _NS * _RPW

_BN = 1000


def _sc_aggregate(xin, src, dst):
    mesh = plsc.VectorSubcoreMesh(core_axis_name="c", subcore_axis_name="s")

    @functools.partial(
        pl.kernel,
        out_type=jax.ShapeDtypeStruct((_NC * _N, _H), jnp.float32),
        mesh=mesh,
        scratch_types=[
            pltpu.VMEM((_CHUNK,), jnp.int32),
            pltpu.VMEM((_CHUNK,), jnp.int32),
            pltpu.VMEM((_TAIL,), jnp.int32),
            pltpu.VMEM((_TAIL,), jnp.int32),
            pltpu.VMEM((_CHUNK, _H), jnp.float32),
            pltpu.VMEM_SHARED((_N, _H), jnp.float32),
            pltpu.SemaphoreType.DMA,
        ],
    )
    def agg(x_hbm, src_hbm, dst_hbm, out_hbm,
            src_v, dst_v, srct_v, dstt_v, rows_v, acc_sh, sem):
        c = lax.axis_index("c")
        s = lax.axis_index("s")

        def zero_body(i, _):
            for j in range(_H // 16):
                rows_v[i, pl.ds(j * 16, 16)] = jnp.zeros((16,), jnp.float32)
            return 0
        lax.fori_loop(0, _CHUNK, zero_body, 0)
        r0 = s * _RPW
        for k in range(4):
            pltpu.sync_copy(rows_v.at[pl.ds(0, _CHUNK)],
                            acc_sh.at[pl.ds(r0 + k * _CHUNK, _CHUNK)])
        pltpu.sync_copy(rows_v.at[pl.ds(0, _RPW - 4 * _CHUNK)],
                        acc_sh.at[pl.ds(r0 + 4 * _CHUNK, _RPW - 4 * _CHUNK)])

        @pl.when(s == _NS - 1)
        def _():
            pltpu.sync_copy(rows_v.at[pl.ds(0, _RTAIL)],
                            acc_sh.at[pl.ds(_NS * _RPW, _RTAIL)])
        plsc.subcore_barrier()

        ebase = c * _EPC + s * _EPW

        def edge_body(t, _):
            e0 = ebase + t * _CHUNK
            pltpu.sync_copy(src_hbm.at[pl.ds(e0, _CHUNK)], src_v)
            pltpu.sync_copy(dst_hbm.at[pl.ds(e0, _CHUNK)], dst_v)
            pltpu.async_copy(x_hbm.at[src_v], rows_v, sem).wait()
            pltpu.sync_copy(rows_v, acc_sh.at[dst_v], add=True)
            return 0
        lax.fori_loop(0, _NFULL, edge_body, 0)

        et = ebase + _NFULL * _CHUNK
        pltpu.sync_copy(src_hbm.at[pl.ds(et, _TAIL)], srct_v)
        pltpu.sync_copy(dst_hbm.at[pl.ds(et, _TAIL)], dstt_v)
        pltpu.async_copy(x_hbm.at[srct_v], rows_v.at[pl.ds(0, _TAIL)], sem).wait()
        pltpu.sync_copy(rows_v.at[pl.ds(0, _TAIL)], acc_sh.at[dstt_v], add=True)

        plsc.subcore_barrier()
        pltpu.sync_copy(acc_sh.at[pl.ds(r0, _RPW)],
                        out_hbm.at[pl.ds(c * _N + r0, _RPW)])

        @pl.when(s == _NS - 1)
        def _():
            pltpu.sync_copy(acc_sh.at[pl.ds(_NS * _RPW, _RTAIL)],
                            out_hbm.at[pl.ds(c * _N + _NS * _RPW, _RTAIL)])

    return agg(xin, src, dst)


def _gin_dense(z, agg0, agg1, wm, bias_row, eps_ref):
    return jnp.maximum(
        lax.dot_general((1.0 + eps_ref[0, 0]) * z + agg0 + agg1, wm,
                        (((1,), (1,)), ((), ())),
                        preferred_element_type=jnp.float32) + bias_row, 0.0)


def _tc_layer1(xin, parts, W, mask, b, eps):
    def body(eps_ref, x_ref, p0_ref, p1_ref, w_ref, m_ref, b_ref, o_ref):
        wm = w_ref[...] * m_ref[...]
        o_ref[...] = _gin_dense(x_ref[...], p0_ref[...], p1_ref[...], wm,
                                b_ref[...], eps_ref)

    nb = _N // _BN
    return pl.pallas_call(
        body,
        grid=(nb,),
        in_specs=[
            pl.BlockSpec(memory_space=pltpu.SMEM),
            pl.BlockSpec((_BN, _H), lambda i: (i, 0)),
            pl.BlockSpec((_BN, _H), lambda i: (i, 0)),
            pl.BlockSpec((_BN, _H), lambda i: (i + nb, 0)),
            pl.BlockSpec((_H, _H), lambda i: (0, 0)),
            pl.BlockSpec((_H, _H), lambda i: (0, 0)),
            pl.BlockSpec((1, _H), lambda i: (0, 0)),
        ],
        out_specs=pl.BlockSpec((_BN, _H), lambda i: (i, 0)),
        out_shape=jax.ShapeDtypeStruct((_N, _H), jnp.float32),
    )(eps.reshape(1, 1), xin, parts, parts, W, mask, b.reshape(1, _H))


def _tc_layer2_pool(zin, parts, W, mask, b, eps, batch_col, Wp_pad, bp_pad):
    nb = _N // _BN

    def body(eps_ref, z_ref, p0_ref, p1_ref, w_ref, m_ref, b_ref,
             bt_ref, wp_ref, bp_ref, o_ref, pool_acc, cnt_acc):
        i = pl.program_id(0)
        wm = w_ref[...] * m_ref[...]
        z2 = _gin_dense(z_ref[...], p0_ref[...], p1_ref[...], wm,
                        b_ref[...], eps_ref)
        gid = lax.broadcasted_iota(jnp.int32, (_BN, _G), 1)
        onehot = (bt_ref[...] == gid).astype(jnp.float32)

        @pl.when(i == 0)
        def _():
            pool_acc[...] = jnp.zeros_like(pool_acc)
            cnt_acc[...] = jnp.zeros_like(cnt_acc)

        dn = (((0,), (0,)), ((), ()))
        pool_acc[...] += lax.dot_general(onehot, z2, dn,
                                         preferred_element_type=jnp.float32)
        cnt_acc[...] += lax.dot_general(onehot, jnp.ones((_BN, _H), jnp.float32),
                                        dn, preferred_element_type=jnp.float32)

        @pl.when(i == nb - 1)
        def _():
            mean = pool_acc[...] / jnp.maximum(cnt_acc[...], 1.0)
            y = lax.dot_general(mean, wp_ref[...], (((1,), (1,)), ((), ())),
                                preferred_element_type=jnp.float32) + bp_ref[...]
            o_ref[...] = jax.nn.sigmoid(y)

    return pl.pallas_call(
        body,
        grid=(nb,),
        in_specs=[
            pl.BlockSpec(memory_space=pltpu.SMEM),
            pl.BlockSpec((_BN, _H), lambda i: (i, 0)),
            pl.BlockSpec((_BN, _H), lambda i: (i, 0)),
            pl.BlockSpec((_BN, _H), lambda i: (i + nb, 0)),
            pl.BlockSpec((_H, _H), lambda i: (0, 0)),
            pl.BlockSpec((_H, _H), lambda i: (0, 0)),
            pl.BlockSpec((1, _H), lambda i: (0, 0)),
            pl.BlockSpec((_BN, 1), lambda i: (i, 0)),
            pl.BlockSpec((_H, _H), lambda i: (0, 0)),
            pl.BlockSpec((1, _H), lambda i: (0, 0)),
        ],
        out_specs=pl.BlockSpec((_G, _H), lambda i: (0, 0)),
        out_shape=jax.ShapeDtypeStruct((_G, _H), jnp.float32),
        scratch_shapes=[
            pltpu.VMEM((_G, _H), jnp.float32),
            pltpu.VMEM((_G, _H), jnp.float32),
        ],
    )(eps.reshape(1, 1), zin, parts, parts, W, mask, b.reshape(1, _H),
      batch_col, Wp_pad, bp_pad)


def kernel(x, edge_index, batch, weights, W1, b1, eps1, W2, b2, eps2, Wp, bp):
    src = edge_index[0]
    dst = edge_index[1]
    batch_col = batch.reshape(_N, 1)
    Wp_pad = jnp.zeros((_H, _H), jnp.float32).at[:_C, :].set(Wp)
    bp_pad = jnp.zeros((1, _H), jnp.float32).at[0, :_C].set(bp)

    parts1 = _sc_aggregate(x, src, dst)
    z1 = _tc_layer1(x, parts1, W1, weights, b1, eps1)
    parts2 = _sc_aggregate(z1, src, dst)
    full = _tc_layer2_pool(z1, parts2, W2, weights, b2, eps2,
                           batch_col, Wp_pad, bp_pad)
    return full[:, :_C]


# speedup vs baseline: 13.3336x; 2.0308x over previous
import functools

import jax
import jax.numpy as jnp
from jax import lax
from jax.experimental import pallas as pl
from jax.experimental.pallas import tpu as pltpu
from jax.experimental.pallas import tpu_sc as plsc

_N = 10000
_E = 320000
_H = 128
_G = 128
_C = 10

_NC = 2
_NS = 16
_EPC = _E // _NC
_EPW = _E // (_NC * _NS)
_CHUNK = 128
_RPW = 624
_RTAIL = _N - _NS * _RPW
_NW = _NC * _NS
_EPW2 = 9984
_PAIRS = 39
_XBASE = _NW * _EPW2

_BN = 1000


def _sc_aggregate(xin, src, dst):
    mesh = plsc.VectorSubcoreMesh(core_axis_name="c", subcore_axis_name="s")

    @functools.partial(
        pl.kernel,
        out_type=jax.ShapeDtypeStruct((_NC * _N, _H), jnp.float32),
        mesh=mesh,
        scratch_types=[
            pltpu.VMEM((_EPW2,), jnp.int32),
            pltpu.VMEM((_CHUNK,), jnp.int32),
            pltpu.VMEM((_CHUNK,), jnp.int32),
            pltpu.VMEM((_CHUNK,), jnp.int32),
            pltpu.VMEM((_CHUNK,), jnp.int32),
            pltpu.VMEM((_CHUNK, _H), jnp.float32),
            pltpu.VMEM((_CHUNK, _H), jnp.float32),
            pltpu.VMEM_SHARED((_N, _H), jnp.float32),
            pltpu.SemaphoreType.DMA,
            pltpu.SemaphoreType.DMA,
            pltpu.SemaphoreType.DMA,
            pltpu.SemaphoreType.DMA,
            pltpu.SemaphoreType.DMA,
        ],
    )
    def agg(x_hbm, src_hbm, dst_hbm, out_hbm,
            srcs_v, dsta_v, dstb_v, srcx_v, dstx_v, rows_a, rows_b, acc_sh,
            sem_s, sem_ga, sem_gb, sem_da, sem_db):
        c = lax.axis_index("c")
        s = lax.axis_index("s")
        w = c * _NS + s
        ebase = w * _EPW2

        stage = pltpu.async_copy(src_hbm.at[pl.ds(ebase, _EPW2)], srcs_v, sem_s)

        def zero_body(i, _):
            for j in range(_H // 16):
                rows_a[i, pl.ds(j * 16, 16)] = jnp.zeros((16,), jnp.float32)
            return 0
        lax.fori_loop(0, _CHUNK, zero_body, 0)
        r0 = s * _RPW
        for k in range(4):
            pltpu.sync_copy(rows_a.at[pl.ds(0, _CHUNK)],
                            acc_sh.at[pl.ds(r0 + k * _CHUNK, _CHUNK)])
        pltpu.sync_copy(rows_a.at[pl.ds(0, _RPW - 4 * _CHUNK)],
                        acc_sh.at[pl.ds(r0 + 4 * _CHUNK, _RPW - 4 * _CHUNK)])

        @pl.when(s == _NS - 1)
        def _():
            pltpu.sync_copy(rows_a.at[pl.ds(0, _RTAIL)],
                            acc_sh.at[pl.ds(_NS * _RPW, _RTAIL)])
        stage.wait()
        plsc.subcore_barrier()

        def start_chunk(j, rows, dstv, sem_g, sem_d):
            pltpu.async_copy(dst_hbm.at[pl.ds(ebase + j * _CHUNK, _CHUNK)],
                             dstv, sem_d)
            pltpu.async_copy(x_hbm.at[srcs_v.at[pl.ds(j * _CHUNK, _CHUNK)]],
                             rows, sem_g)

        def finish_chunk(j, rows, dstv, sem_g, sem_d):
            pltpu.make_async_copy(
                x_hbm.at[srcs_v.at[pl.ds(j * _CHUNK, _CHUNK)]],
                rows, sem_g).wait()
            pltpu.make_async_copy(
                dst_hbm.at[pl.ds(ebase + j * _CHUNK, _CHUNK)],
                dstv, sem_d).wait()
            pltpu.sync_copy(rows, acc_sh.at[dstv], add=True)

        start_chunk(0, rows_a, dsta_v, sem_ga, sem_da)

        def body(t, _):
            start_chunk(2 * t + 1, rows_b, dstb_v, sem_gb, sem_db)
            finish_chunk(2 * t, rows_a, dsta_v, sem_ga, sem_da)

            @pl.when(t < _PAIRS - 1)
            def _():
                start_chunk(2 * t + 2, rows_a, dsta_v, sem_ga, sem_da)
            finish_chunk(2 * t + 1, rows_b, dstb_v, sem_gb, sem_db)
            return 0
        lax.fori_loop(0, _PAIRS, body, 0)

        @pl.when(w < 4)
        def _():
            e0 = _XBASE + w * _CHUNK
            pltpu.sync_copy(src_hbm.at[pl.ds(e0, _CHUNK)], srcx_v)
            pltpu.sync_copy(dst_hbm.at[pl.ds(e0, _CHUNK)], dstx_v)
            pltpu.async_copy(x_hbm.at[srcx_v], rows_a, sem_ga).wait()
            pltpu.sync_copy(rows_a, acc_sh.at[dstx_v], add=True)

        plsc.subcore_barrier()
        pltpu.sync_copy(acc_sh.at[pl.ds(r0, _RPW)],
                        out_hbm.at[pl.ds(c * _N + r0, _RPW)])

        @pl.when(s == _NS - 1)
        def _():
            pltpu.sync_copy(acc_sh.at[pl.ds(_NS * _RPW, _RTAIL)],
                            out_hbm.at[pl.ds(c * _N + _NS * _RPW, _RTAIL)])

    return agg(xin, src, dst)


def _gin_dense(z, agg0, agg1, wm, bias_row, eps_ref):
    return jnp.maximum(
        lax.dot_general((1.0 + eps_ref[0, 0]) * z + agg0 + agg1, wm,
                        (((1,), (1,)), ((), ())),
                        preferred_element_type=jnp.float32) + bias_row, 0.0)


def _tc_layer1(xin, parts, W, mask, b, eps):
    def body(eps_ref, x_ref, p0_ref, p1_ref, w_ref, m_ref, b_ref, o_ref):
        wm = w_ref[...] * m_ref[...]
        o_ref[...] = _gin_dense(x_ref[...], p0_ref[...], p1_ref[...], wm,
                                b_ref[...], eps_ref)

    nb = _N // _BN
    return pl.pallas_call(
        body,
        grid=(nb,),
        in_specs=[
            pl.BlockSpec(memory_space=pltpu.SMEM),
            pl.BlockSpec((_BN, _H), lambda i: (i, 0)),
            pl.BlockSpec((_BN, _H), lambda i: (i, 0)),
            pl.BlockSpec((_BN, _H), lambda i: (i + nb, 0)),
            pl.BlockSpec((_H, _H), lambda i: (0, 0)),
            pl.BlockSpec((_H, _H), lambda i: (0, 0)),
            pl.BlockSpec((1, _H), lambda i: (0, 0)),
        ],
        out_specs=pl.BlockSpec((_BN, _H), lambda i: (i, 0)),
        out_shape=jax.ShapeDtypeStruct((_N, _H), jnp.float32),
    )(eps.reshape(1, 1), xin, parts, parts, W, mask, b.reshape(1, _H))


def _tc_layer2_pool(zin, parts, W, mask, b, eps, batch_col, Wp_pad, bp_pad):
    nb = _N // _BN

    def body(eps_ref, z_ref, p0_ref, p1_ref, w_ref, m_ref, b_ref,
             bt_ref, wp_ref, bp_ref, o_ref, pool_acc, cnt_acc):
        i = pl.program_id(0)
        wm = w_ref[...] * m_ref[...]
        z2 = _gin_dense(z_ref[...], p0_ref[...], p1_ref[...], wm,
                        b_ref[...], eps_ref)
        gid = lax.broadcasted_iota(jnp.int32, (_BN, _G), 1)
        onehot = (bt_ref[...] == gid).astype(jnp.float32)

        @pl.when(i == 0)
        def _():
            pool_acc[...] = jnp.zeros_like(pool_acc)
            cnt_acc[...] = jnp.zeros_like(cnt_acc)

        dn = (((0,), (0,)), ((), ()))
        pool_acc[...] += lax.dot_general(onehot, z2, dn,
                                         preferred_element_type=jnp.float32)
        cnt_acc[...] += lax.dot_general(onehot, jnp.ones((_BN, _H), jnp.float32),
                                        dn, preferred_element_type=jnp.float32)

        @pl.when(i == nb - 1)
        def _():
            mean = pool_acc[...] / jnp.maximum(cnt_acc[...], 1.0)
            y = lax.dot_general(mean, wp_ref[...], (((1,), (1,)), ((), ())),
                                preferred_element_type=jnp.float32) + bp_ref[...]
            o_ref[...] = jax.nn.sigmoid(y)

    return pl.pallas_call(
        body,
        grid=(nb,),
        in_specs=[
            pl.BlockSpec(memory_space=pltpu.SMEM),
            pl.BlockSpec((_BN, _H), lambda i: (i, 0)),
            pl.BlockSpec((_BN, _H), lambda i: (i, 0)),
            pl.BlockSpec((_BN, _H), lambda i: (i + nb, 0)),
            pl.BlockSpec((_H, _H), lambda i: (0, 0)),
            pl.BlockSpec((_H, _H), lambda i: (0, 0)),
            pl.BlockSpec((1, _H), lambda i: (0, 0)),
            pl.BlockSpec((_BN, 1), lambda i: (i, 0)),
            pl.BlockSpec((_H, _H), lambda i: (0, 0)),
            pl.BlockSpec((1, _H), lambda i: (0, 0)),
        ],
        out_specs=pl.BlockSpec((_G, _H), lambda i: (0, 0)),
        out_shape=jax.ShapeDtypeStruct((_G, _H), jnp.float32),
        scratch_shapes=[
            pltpu.VMEM((_G, _H), jnp.float32),
            pltpu.VMEM((_G, _H), jnp.float32),
        ],
    )(eps.reshape(1, 1), zin, parts, parts, W, mask, b.reshape(1, _H),
      batch_col, Wp_pad, bp_pad)


def kernel(x, edge_index, batch, weights, W1, b1, eps1, W2, b2, eps2, Wp, bp):
    src = edge_index[0]
    dst = edge_index[1]
    batch_col = batch.reshape(_N, 1)
    Wp_pad = jnp.zeros((_H, _H), jnp.float32).at[:_C, :].set(Wp)
    bp_pad = jnp.zeros((1, _H), jnp.float32).at[0, :_C].set(bp)

    parts1 = _sc_aggregate(x, src, dst)
    z1 = _tc_layer1(x, parts1, W1, weights, b1, eps1)
    parts2 = _sc_aggregate(z1, src, dst)
    full = _tc_layer2_pool(z1, parts2, W2, weights, b2, eps2,
                           batch_col, Wp_pad, bp_pad)
    return full[:, :_C]


# speedup vs baseline: 14.0735x; 1.0555x over previous
import functools

import jax
import jax.numpy as jnp
from jax import lax
from jax.experimental import pallas as pl
from jax.experimental.pallas import tpu as pltpu
from jax.experimental.pallas import tpu_sc as plsc

_N = 10000
_E = 320000
_H = 128
_G = 128
_C = 10

_NC = 2
_NS = 16
_EPC = _E // _NC
_EPW = _E // (_NC * _NS)
_CHUNK = 104
_RPW = 624
_RTAIL = _N - _NS * _RPW
_NW = _NC * _NS
_EPW2 = 9984
_NCH = _EPW2 // _CHUNK
_NBUF = 3
_XBASE = _NW * _EPW2
_XCH = 64

_BN = 1000


def _sc_aggregate(xin, src, dst):
    mesh = plsc.VectorSubcoreMesh(core_axis_name="c", subcore_axis_name="s")

    @functools.partial(
        pl.kernel,
        out_type=jax.ShapeDtypeStruct((_NC * _N, _H), jnp.float32),
        mesh=mesh,
        scratch_types=[
            pltpu.VMEM((_EPW2,), jnp.int32),
            pltpu.VMEM((_NBUF, _CHUNK), jnp.int32),
            pltpu.VMEM((_NBUF, _CHUNK, _H), jnp.float32),
            pltpu.VMEM((_XCH,), jnp.int32),
            pltpu.VMEM((_XCH,), jnp.int32),
            pltpu.VMEM_SHARED((_N, _H), jnp.float32),
            pltpu.SemaphoreType.DMA,
            pltpu.SemaphoreType.DMA((_NBUF,)),
            pltpu.SemaphoreType.DMA((_NBUF,)),
            pltpu.SemaphoreType.DMA((_NBUF,)),
        ],
    )
    def agg(x_hbm, src_hbm, dst_hbm, out_hbm,
            srcs_v, dring, rows, srcx_v, dstx_v, acc_sh,
            sem_s, gsem, dsem, ssem):
        c = lax.axis_index("c")
        s = lax.axis_index("s")
        w = c * _NS + s
        ebase = w * _EPW2

        stage = pltpu.async_copy(src_hbm.at[pl.ds(ebase, _EPW2)], srcs_v, sem_s)

        def zero_body(i, _):
            for j in range(_H // 16):
                rows[0, i, pl.ds(j * 16, 16)] = jnp.zeros((16,), jnp.float32)
            return 0
        lax.fori_loop(0, _CHUNK, zero_body, 0)
        r0 = s * _RPW
        for k in range(_RPW // _CHUNK):
            pltpu.sync_copy(rows.at[0].at[pl.ds(0, _CHUNK)],
                            acc_sh.at[pl.ds(r0 + k * _CHUNK, _CHUNK)])

        @pl.when(s == _NS - 1)
        def _():
            pltpu.sync_copy(rows.at[0].at[pl.ds(0, _RTAIL)],
                            acc_sh.at[pl.ds(_NS * _RPW, _RTAIL)])
        stage.wait()
        plsc.subcore_barrier()

        def fire_chunk(j, b):
            pltpu.async_copy(dst_hbm.at[pl.ds(ebase + j * _CHUNK, _CHUNK)],
                             dring.at[b], dsem.at[b])
            pltpu.async_copy(x_hbm.at[srcs_v.at[pl.ds(j * _CHUNK, _CHUNK)]],
                             rows.at[b], gsem.at[b])

        def wait_gather(j, b):
            pltpu.make_async_copy(
                x_hbm.at[srcs_v.at[pl.ds(j * _CHUNK, _CHUNK)]],
                rows.at[b], gsem.at[b]).wait()
            pltpu.make_async_copy(
                dst_hbm.at[pl.ds(ebase + j * _CHUNK, _CHUNK)],
                dring.at[b], dsem.at[b]).wait()

        def wait_scatter(b):
            pltpu.make_async_copy(rows.at[b], acc_sh.at[dring.at[b]],
                                  ssem.at[b]).wait()

        for b in range(_NBUF - 1):
            fire_chunk(b, b)

        def body(t, _):
            for b in range(_NBUF):
                j = t * _NBUF + b
                wait_gather(j, b)
                pltpu.async_copy(rows.at[b], acc_sh.at[dring.at[b]],
                                 ssem.at[b], add=True)
                b2 = (b + _NBUF - 1) % _NBUF

                @pl.when(j >= 1)
                def _():
                    wait_scatter(b2)

                @pl.when(j + 2 < _NCH)
                def _():
                    fire_chunk(j + 2, b2)
            return 0
        lax.fori_loop(0, _NCH // _NBUF, body, 0)

        wait_scatter((_NCH - 1) % _NBUF)

        @pl.when(w < 8)
        def _():
            e0 = _XBASE + w * _XCH
            pltpu.sync_copy(src_hbm.at[pl.ds(e0, _XCH)], srcx_v)
            pltpu.sync_copy(dst_hbm.at[pl.ds(e0, _XCH)], dstx_v)
            pltpu.async_copy(x_hbm.at[srcx_v],
                             rows.at[0].at[pl.ds(0, _XCH)], gsem.at[0]).wait()
            pltpu.sync_copy(rows.at[0].at[pl.ds(0, _XCH)],
                            acc_sh.at[dstx_v], add=True)

        plsc.subcore_barrier()
        pltpu.sync_copy(acc_sh.at[pl.ds(r0, _RPW)],
                        out_hbm.at[pl.ds(c * _N + r0, _RPW)])

        @pl.when(s == _NS - 1)
        def _():
            pltpu.sync_copy(acc_sh.at[pl.ds(_NS * _RPW, _RTAIL)],
                            out_hbm.at[pl.ds(c * _N + _NS * _RPW, _RTAIL)])

    return agg(xin, src, dst)


def _gin_dense(z, agg0, agg1, wm, bias_row, eps_ref):
    return jnp.maximum(
        lax.dot_general((1.0 + eps_ref[0, 0]) * z + agg0 + agg1, wm,
                        (((1,), (1,)), ((), ())),
                        preferred_element_type=jnp.float32) + bias_row, 0.0)


def _tc_layer1(xin, parts, W, mask, b, eps):
    def body(eps_ref, x_ref, p0_ref, p1_ref, w_ref, m_ref, b_ref, o_ref):
        wm = w_ref[...] * m_ref[...]
        o_ref[...] = _gin_dense(x_ref[...], p0_ref[...], p1_ref[...], wm,
                                b_ref[...], eps_ref)

    nb = _N // _BN
    return pl.pallas_call(
        body,
        grid=(nb,),
        in_specs=[
            pl.BlockSpec(memory_space=pltpu.SMEM),
            pl.BlockSpec((_BN, _H), lambda i: (i, 0)),
            pl.BlockSpec((_BN, _H), lambda i: (i, 0)),
            pl.BlockSpec((_BN, _H), lambda i: (i + nb, 0)),
            pl.BlockSpec((_H, _H), lambda i: (0, 0)),
            pl.BlockSpec((_H, _H), lambda i: (0, 0)),
            pl.BlockSpec((1, _H), lambda i: (0, 0)),
        ],
        out_specs=pl.BlockSpec((_BN, _H), lambda i: (i, 0)),
        out_shape=jax.ShapeDtypeStruct((_N, _H), jnp.float32),
    )(eps.reshape(1, 1), xin, parts, parts, W, mask, b.reshape(1, _H))


def _tc_layer2_pool(zin, parts, W, mask, b, eps, batch_col, Wp_pad, bp_pad):
    nb = _N // _BN

    def body(eps_ref, z_ref, p0_ref, p1_ref, w_ref, m_ref, b_ref,
             bt_ref, wp_ref, bp_ref, o_ref, pool_acc, cnt_acc):
        i = pl.program_id(0)
        wm = w_ref[...] * m_ref[...]
        z2 = _gin_dense(z_ref[...], p0_ref[...], p1_ref[...], wm,
                        b_ref[...], eps_ref)
        gid = lax.broadcasted_iota(jnp.int32, (_BN, _G), 1)
        onehot = (bt_ref[...] == gid).astype(jnp.float32)

        @pl.when(i == 0)
        def _():
            pool_acc[...] = jnp.zeros_like(pool_acc)
            cnt_acc[...] = jnp.zeros_like(cnt_acc)

        dn = (((0,), (0,)), ((), ()))
        pool_acc[...] += lax.dot_general(onehot, z2, dn,
                                         preferred_element_type=jnp.float32)
        cnt_acc[...] += lax.dot_general(onehot, jnp.ones((_BN, _H), jnp.float32),
                                        dn, preferred_element_type=jnp.float32)

        @pl.when(i == nb - 1)
        def _():
            mean = pool_acc[...] / jnp.maximum(cnt_acc[...], 1.0)
            y = lax.dot_general(mean, wp_ref[...], (((1,), (1,)), ((), ())),
                                preferred_element_type=jnp.float32) + bp_ref[...]
            o_ref[...] = jax.nn.sigmoid(y)

    return pl.pallas_call(
        body,
        grid=(nb,),
        in_specs=[
            pl.BlockSpec(memory_space=pltpu.SMEM),
            pl.BlockSpec((_BN, _H), lambda i: (i, 0)),
            pl.BlockSpec((_BN, _H), lambda i: (i, 0)),
            pl.BlockSpec((_BN, _H), lambda i: (i + nb, 0)),
            pl.BlockSpec((_H, _H), lambda i: (0, 0)),
            pl.BlockSpec((_H, _H), lambda i: (0, 0)),
            pl.BlockSpec((1, _H), lambda i: (0, 0)),
            pl.BlockSpec((_BN, 1), lambda i: (i, 0)),
            pl.BlockSpec((_H, _H), lambda i: (0, 0)),
            pl.BlockSpec((1, _H), lambda i: (0, 0)),
        ],
        out_specs=pl.BlockSpec((_G, _H), lambda i: (0, 0)),
        out_shape=jax.ShapeDtypeStruct((_G, _H), jnp.float32),
        scratch_shapes=[
            pltpu.VMEM((_G, _H), jnp.float32),
            pltpu.VMEM((_G, _H), jnp.float32),
        ],
    )(eps.reshape(1, 1), zin, parts, parts, W, mask, b.reshape(1, _H),
      batch_col, Wp_pad, bp_pad)


def kernel(x, edge_index, batch, weights, W1, b1, eps1, W2, b2, eps2, Wp, bp):
    src = edge_index[0]
    dst = edge_index[1]
    batch_col = batch.reshape(_N, 1)
    Wp_pad = jnp.zeros((_H, _H), jnp.float32).at[:_C, :].set(Wp)
    bp_pad = jnp.zeros((1, _H), jnp.float32).at[0, :_C].set(bp)

    parts1 = _sc_aggregate(x, src, dst)
    z1 = _tc_layer1(x, parts1, W1, weights, b1, eps1)
    parts2 = _sc_aggregate(z1, src, dst)
    full = _tc_layer2_pool(z1, parts2, W2, weights, b2, eps2,
                           batch_col, Wp_pad, bp_pad)
    return full[:, :_C]


# speedup vs baseline: 14.4209x; 1.0247x over previous
import functools

import jax
import jax.numpy as jnp
from jax import lax
from jax.experimental import pallas as pl
from jax.experimental.pallas import tpu as pltpu
from jax.experimental.pallas import tpu_sc as plsc

_N = 10000
_E = 320000
_H = 128
_G = 128
_C = 10

_NC = 2
_NS = 16
_EPC = _E // _NC
_EPW = _E // (_NC * _NS)
_CHUNK = 104
_RPW = 624
_RTAIL = _N - _NS * _RPW
_NW = _NC * _NS
_EPW2 = 9984
_NCH = _EPW2 // _CHUNK
_NBUF = 3
_XBASE = _NW * _EPW2
_XCH = 64

_BN = 2000


def _sc_aggregate(xin, src, dst):
    mesh = plsc.VectorSubcoreMesh(core_axis_name="c", subcore_axis_name="s")

    @functools.partial(
        pl.kernel,
        out_type=jax.ShapeDtypeStruct((_NC * _N, _H), jnp.float32),
        mesh=mesh,
        scratch_types=[
            pltpu.VMEM((_EPW2,), jnp.int32),
            pltpu.VMEM((_NBUF, _CHUNK), jnp.int32),
            pltpu.VMEM((_NBUF, _CHUNK, _H), jnp.float32),
            pltpu.VMEM((_XCH,), jnp.int32),
            pltpu.VMEM((_XCH,), jnp.int32),
            pltpu.VMEM_SHARED((_N, _H), jnp.float32),
            pltpu.SemaphoreType.DMA,
            pltpu.SemaphoreType.DMA((_NBUF,)),
            pltpu.SemaphoreType.DMA((_NBUF,)),
            pltpu.SemaphoreType.DMA((_NBUF,)),
        ],
    )
    def agg(x_hbm, src_hbm, dst_hbm, out_hbm,
            srcs_v, dring, rows, srcx_v, dstx_v, acc_sh,
            sem_s, gsem, dsem, ssem):
        c = lax.axis_index("c")
        s = lax.axis_index("s")
        w = c * _NS + s
        ebase = w * _EPW2

        stage = pltpu.async_copy(src_hbm.at[pl.ds(ebase, _EPW2)], srcs_v, sem_s)

        def zero_body(i, _):
            for j in range(_H // 16):
                rows[0, i, pl.ds(j * 16, 16)] = jnp.zeros((16,), jnp.float32)
            return 0
        lax.fori_loop(0, _CHUNK, zero_body, 0)
        r0 = s * _RPW
        for k in range(_RPW // _CHUNK):
            pltpu.sync_copy(rows.at[0].at[pl.ds(0, _CHUNK)],
                            acc_sh.at[pl.ds(r0 + k * _CHUNK, _CHUNK)])

        @pl.when(s == _NS - 1)
        def _():
            pltpu.sync_copy(rows.at[0].at[pl.ds(0, _RTAIL)],
                            acc_sh.at[pl.ds(_NS * _RPW, _RTAIL)])
        stage.wait()
        plsc.subcore_barrier()

        def fire_chunk(j, b):
            pltpu.async_copy(dst_hbm.at[pl.ds(ebase + j * _CHUNK, _CHUNK)],
                             dring.at[b], dsem.at[b])
            pltpu.async_copy(x_hbm.at[srcs_v.at[pl.ds(j * _CHUNK, _CHUNK)]],
                             rows.at[b], gsem.at[b])

        def wait_gather(j, b):
            pltpu.make_async_copy(
                x_hbm.at[srcs_v.at[pl.ds(j * _CHUNK, _CHUNK)]],
                rows.at[b], gsem.at[b]).wait()
            pltpu.make_async_copy(
                dst_hbm.at[pl.ds(ebase + j * _CHUNK, _CHUNK)],
                dring.at[b], dsem.at[b]).wait()

        def wait_scatter(b):
            pltpu.make_async_copy(rows.at[b], acc_sh.at[dring.at[b]],
                                  ssem.at[b]).wait()

        for b in range(_NBUF - 1):
            fire_chunk(b, b)

        def body(t, _):
            for b in range(_NBUF):
                j = t * _NBUF + b
                wait_gather(j, b)
                pltpu.async_copy(rows.at[b], acc_sh.at[dring.at[b]],
                                 ssem.at[b], add=True)
                b2 = (b + _NBUF - 1) % _NBUF

                @pl.when(j >= 1)
                def _():
                    wait_scatter(b2)

                @pl.when(j + 2 < _NCH)
                def _():
                    fire_chunk(j + 2, b2)
            return 0
        lax.fori_loop(0, _NCH // _NBUF, body, 0)

        wait_scatter((_NCH - 1) % _NBUF)

        @pl.when(w < 8)
        def _():
            e0 = _XBASE + w * _XCH
            pltpu.sync_copy(src_hbm.at[pl.ds(e0, _XCH)], srcx_v)
            pltpu.sync_copy(dst_hbm.at[pl.ds(e0, _XCH)], dstx_v)
            pltpu.async_copy(x_hbm.at[srcx_v],
                             rows.at[0].at[pl.ds(0, _XCH)], gsem.at[0]).wait()
            pltpu.sync_copy(rows.at[0].at[pl.ds(0, _XCH)],
                            acc_sh.at[dstx_v], add=True)

        plsc.subcore_barrier()
        pltpu.sync_copy(acc_sh.at[pl.ds(r0, _RPW)],
                        out_hbm.at[pl.ds(c * _N + r0, _RPW)])

        @pl.when(s == _NS - 1)
        def _():
            pltpu.sync_copy(acc_sh.at[pl.ds(_NS * _RPW, _RTAIL)],
                            out_hbm.at[pl.ds(c * _N + _NS * _RPW, _RTAIL)])

    return agg(xin, src, dst)


def _gin_dense(z, agg0, agg1, wm, bias_row, eps_ref):
    return jnp.maximum(
        lax.dot_general((1.0 + eps_ref[0, 0]) * z + agg0 + agg1, wm,
                        (((1,), (1,)), ((), ())),
                        preferred_element_type=jnp.float32) + bias_row, 0.0)


def _tc_layer1(xin, parts, W, mask, b, eps):
    def body(eps_ref, x_ref, p0_ref, p1_ref, w_ref, m_ref, b_ref, o_ref):
        wm = w_ref[...] * m_ref[...]
        o_ref[...] = _gin_dense(x_ref[...], p0_ref[...], p1_ref[...], wm,
                                b_ref[...], eps_ref)

    nb = _N // _BN
    return pl.pallas_call(
        body,
        grid=(nb,),
        in_specs=[
            pl.BlockSpec(memory_space=pltpu.SMEM),
            pl.BlockSpec((_BN, _H), lambda i: (i, 0)),
            pl.BlockSpec((_BN, _H), lambda i: (i, 0)),
            pl.BlockSpec((_BN, _H), lambda i: (i + nb, 0)),
            pl.BlockSpec((_H, _H), lambda i: (0, 0)),
            pl.BlockSpec((_H, _H), lambda i: (0, 0)),
            pl.BlockSpec((1, _H), lambda i: (0, 0)),
        ],
        out_specs=pl.BlockSpec((_BN, _H), lambda i: (i, 0)),
        out_shape=jax.ShapeDtypeStruct((_N, _H), jnp.float32),
    )(eps.reshape(1, 1), xin, parts, parts, W, mask, b.reshape(1, _H))


def _tc_layer2_pool(zin, parts, W, mask, b, eps, batch_col, Wp, bp):
    nb = _N // _BN

    def body(eps_ref, z_ref, p0_ref, p1_ref, w_ref, m_ref, b_ref,
             bt_ref, wp_ref, bp_ref, o_ref, pool_acc, cnt_acc):
        i = pl.program_id(0)
        wm = w_ref[...] * m_ref[...]
        z2 = _gin_dense(z_ref[...], p0_ref[...], p1_ref[...], wm,
                        b_ref[...], eps_ref)
        gid = lax.broadcasted_iota(jnp.int32, (_BN, _G), 1)
        onehot = (bt_ref[...] == gid).astype(jnp.float32)

        @pl.when(i == 0)
        def _():
            pool_acc[...] = jnp.zeros_like(pool_acc)
            cnt_acc[...] = jnp.zeros_like(cnt_acc)

        dn = (((0,), (0,)), ((), ()))
        pool_acc[...] += lax.dot_general(onehot, z2, dn,
                                         preferred_element_type=jnp.float32)
        cnt_acc[...] += lax.dot_general(onehot, jnp.ones((_BN, _H), jnp.float32),
                                        dn, preferred_element_type=jnp.float32)

        @pl.when(i == nb - 1)
        def _():
            mean = pool_acc[...] / jnp.maximum(cnt_acc[...], 1.0)
            y = lax.dot_general(mean, wp_ref[...], (((1,), (1,)), ((), ())),
                                preferred_element_type=jnp.float32) + bp_ref[...]
            o_ref[...] = jax.nn.sigmoid(y)

    return pl.pallas_call(
        body,
        grid=(nb,),
        in_specs=[
            pl.BlockSpec(memory_space=pltpu.SMEM),
            pl.BlockSpec((_BN, _H), lambda i: (i, 0)),
            pl.BlockSpec((_BN, _H), lambda i: (i, 0)),
            pl.BlockSpec((_BN, _H), lambda i: (i + nb, 0)),
            pl.BlockSpec((_H, _H), lambda i: (0, 0)),
            pl.BlockSpec((_H, _H), lambda i: (0, 0)),
            pl.BlockSpec((1, _H), lambda i: (0, 0)),
            pl.BlockSpec((_BN, 1), lambda i: (i, 0)),
            pl.BlockSpec((_C, _H), lambda i: (0, 0)),
            pl.BlockSpec((1, _C), lambda i: (0, 0)),
        ],
        out_specs=pl.BlockSpec((_G, _C), lambda i: (0, 0)),
        out_shape=jax.ShapeDtypeStruct((_G, _C), jnp.float32),
        scratch_shapes=[
            pltpu.VMEM((_G, _H), jnp.float32),
            pltpu.VMEM((_G, _H), jnp.float32),
        ],
    )(eps.reshape(1, 1), zin, parts, parts, W, mask, b.reshape(1, _H),
      batch_col, Wp, bp.reshape(1, _C))


def kernel(x, edge_index, batch, weights, W1, b1, eps1, W2, b2, eps2, Wp, bp):
    src = edge_index[0]
    dst = edge_index[1]
    batch_col = batch.reshape(_N, 1)
    parts1 = _sc_aggregate(x, src, dst)
    z1 = _tc_layer1(x, parts1, W1, weights, b1, eps1)
    parts2 = _sc_aggregate(z1, src, dst)
    return _tc_layer2_pool(z1, parts2, W2, weights, b2, eps2,
                           batch_col, Wp, bp)


# speedup vs baseline: 14.4274x; 1.0005x over previous
import functools

import jax
import jax.numpy as jnp
from jax import lax
from jax.experimental import pallas as pl
from jax.experimental.pallas import tpu as pltpu
from jax.experimental.pallas import tpu_sc as plsc

_N = 10000
_E = 320000
_H = 128
_G = 128
_C = 10

_NC = 2
_NS = 16
_EPC = _E // _NC
_EPW = _E // (_NC * _NS)
_CHUNK = 104
_RPW = 624
_RTAIL = _N - _NS * _RPW
_NW = _NC * _NS
_EPW2 = 9984
_NCH = _EPW2 // _CHUNK
_NBUF = 3
_XBASE = _NW * _EPW2
_XCH = 64

_BN = 2000


def _sc_aggregate(xin, src, dst):
    mesh = plsc.VectorSubcoreMesh(core_axis_name="c", subcore_axis_name="s")

    @functools.partial(
        pl.kernel,
        out_type=jax.ShapeDtypeStruct((_NC * _N, _H), jnp.float32),
        mesh=mesh,
        scratch_types=[
            pltpu.VMEM((_EPW2,), jnp.int32),
            pltpu.VMEM((_NBUF, _CHUNK), jnp.int32),
            pltpu.VMEM((_NBUF, _CHUNK, _H), jnp.float32),
            pltpu.VMEM((_XCH,), jnp.int32),
            pltpu.VMEM((_XCH,), jnp.int32),
            pltpu.VMEM_SHARED((_N, _H), jnp.float32),
            pltpu.SemaphoreType.DMA,
            pltpu.SemaphoreType.DMA((_NBUF,)),
            pltpu.SemaphoreType.DMA((_NBUF,)),
            pltpu.SemaphoreType.DMA((_NBUF,)),
        ],
    )
    def agg(x_hbm, src_hbm, dst_hbm, out_hbm,
            srcs_v, dring, rows, srcx_v, dstx_v, acc_sh,
            sem_s, gsem, dsem, ssem):
        c = lax.axis_index("c")
        s = lax.axis_index("s")
        w = c * _NS + s
        ebase = w * _EPW2

        def fire_chunk(j, b):
            pltpu.async_copy(dst_hbm.at[pl.ds(ebase + j * _CHUNK, _CHUNK)],
                             dring.at[b], dsem.at[b])
            pltpu.async_copy(x_hbm.at[srcs_v.at[pl.ds(j * _CHUNK, _CHUNK)]],
                             rows.at[b], gsem.at[b])

        def wait_gather(j, b):
            pltpu.make_async_copy(
                x_hbm.at[srcs_v.at[pl.ds(j * _CHUNK, _CHUNK)]],
                rows.at[b], gsem.at[b]).wait()
            pltpu.make_async_copy(
                dst_hbm.at[pl.ds(ebase + j * _CHUNK, _CHUNK)],
                dring.at[b], dsem.at[b]).wait()

        def wait_scatter(b):
            pltpu.make_async_copy(rows.at[b], acc_sh.at[dring.at[b]],
                                  ssem.at[b]).wait()


        stage = pltpu.async_copy(src_hbm.at[pl.ds(ebase, _EPW2)], srcs_v, sem_s)

        def zero_body(i, _):
            for j in range(_H // 16):
                rows[2, i, pl.ds(j * 16, 16)] = jnp.zeros((16,), jnp.float32)
            return 0
        lax.fori_loop(0, _CHUNK, zero_body, 0)
        stage.wait()
        for b in range(_NBUF - 1):
            fire_chunk(b, b)
        r0 = s * _RPW
        for k in range(_RPW // _CHUNK):
            pltpu.async_copy(rows.at[2].at[pl.ds(0, _CHUNK)],
                            acc_sh.at[pl.ds(r0 + k * _CHUNK, _CHUNK)], ssem.at[2])

        @pl.when(s == _NS - 1)
        def _():
            pltpu.async_copy(rows.at[2].at[pl.ds(0, _RTAIL)],
                            acc_sh.at[pl.ds(_NS * _RPW, _RTAIL)], ssem.at[2])
        for k in range(_RPW // _CHUNK):
            pltpu.make_async_copy(rows.at[2].at[pl.ds(0, _CHUNK)],
                                  acc_sh.at[pl.ds(r0 + k * _CHUNK, _CHUNK)],
                                  ssem.at[2]).wait()

        @pl.when(s == _NS - 1)
        def _():
            pltpu.make_async_copy(rows.at[2].at[pl.ds(0, _RTAIL)],
                                  acc_sh.at[pl.ds(_NS * _RPW, _RTAIL)],
                                  ssem.at[2]).wait()
        plsc.subcore_barrier()

        def body(t, _):
            for b in range(_NBUF):
                j = t * _NBUF + b
                wait_gather(j, b)
                pltpu.async_copy(rows.at[b], acc_sh.at[dring.at[b]],
                                 ssem.at[b], add=True)
                b2 = (b + _NBUF - 1) % _NBUF

                @pl.when(j >= 1)
                def _():
                    wait_scatter(b2)

                @pl.when(j + 2 < _NCH)
                def _():
                    fire_chunk(j + 2, b2)
            return 0
        lax.fori_loop(0, _NCH // _NBUF, body, 0)

        wait_scatter((_NCH - 1) % _NBUF)

        @pl.when(w < 8)
        def _():
            e0 = _XBASE + w * _XCH
            pltpu.sync_copy(src_hbm.at[pl.ds(e0, _XCH)], srcx_v)
            pltpu.sync_copy(dst_hbm.at[pl.ds(e0, _XCH)], dstx_v)
            pltpu.async_copy(x_hbm.at[srcx_v],
                             rows.at[0].at[pl.ds(0, _XCH)], gsem.at[0]).wait()
            pltpu.sync_copy(rows.at[0].at[pl.ds(0, _XCH)],
                            acc_sh.at[dstx_v], add=True)

        plsc.subcore_barrier()
        pltpu.sync_copy(acc_sh.at[pl.ds(r0, _RPW)],
                        out_hbm.at[pl.ds(c * _N + r0, _RPW)])

        @pl.when(s == _NS - 1)
        def _():
            pltpu.sync_copy(acc_sh.at[pl.ds(_NS * _RPW, _RTAIL)],
                            out_hbm.at[pl.ds(c * _N + _NS * _RPW, _RTAIL)])

    return agg(xin, src, dst)


def _gin_dense(z, agg0, agg1, wm, bias_row, eps_ref):
    return jnp.maximum(
        lax.dot_general((1.0 + eps_ref[0, 0]) * z + agg0 + agg1, wm,
                        (((1,), (1,)), ((), ())),
                        preferred_element_type=jnp.float32) + bias_row, 0.0)


def _tc_layer1(xin, parts, W, mask, b, eps):
    def body(eps_ref, x_ref, p0_ref, p1_ref, w_ref, m_ref, b_ref, o_ref):
        wm = w_ref[...] * m_ref[...]
        o_ref[...] = _gin_dense(x_ref[...], p0_ref[...], p1_ref[...], wm,
                                b_ref[...], eps_ref)

    nb = _N // _BN
    return pl.pallas_call(
        body,
        grid=(nb,),
        in_specs=[
            pl.BlockSpec(memory_space=pltpu.SMEM),
            pl.BlockSpec((_BN, _H), lambda i: (i, 0)),
            pl.BlockSpec((_BN, _H), lambda i: (i, 0)),
            pl.BlockSpec((_BN, _H), lambda i: (i + nb, 0)),
            pl.BlockSpec((_H, _H), lambda i: (0, 0)),
            pl.BlockSpec((_H, _H), lambda i: (0, 0)),
            pl.BlockSpec((1, _H), lambda i: (0, 0)),
        ],
        out_specs=pl.BlockSpec((_BN, _H), lambda i: (i, 0)),
        out_shape=jax.ShapeDtypeStruct((_N, _H), jnp.float32),
    )(eps.reshape(1, 1), xin, parts, parts, W, mask, b.reshape(1, _H))


def _tc_layer2_pool(zin, parts, W, mask, b, eps, batch_col, Wp, bp):
    nb = _N // _BN

    def body(eps_ref, z_ref, p0_ref, p1_ref, w_ref, m_ref, b_ref,
             bt_ref, wp_ref, bp_ref, o_ref, pool_acc, cnt_acc):
        i = pl.program_id(0)
        wm = w_ref[...] * m_ref[...]
        z2 = _gin_dense(z_ref[...], p0_ref[...], p1_ref[...], wm,
                        b_ref[...], eps_ref)
        gid = lax.broadcasted_iota(jnp.int32, (_BN, _G), 1)
        onehot = (bt_ref[...] == gid).astype(jnp.float32)

        @pl.when(i == 0)
        def _():
            pool_acc[...] = jnp.zeros_like(pool_acc)
            cnt_acc[...] = jnp.zeros_like(cnt_acc)

        dn = (((0,), (0,)), ((), ()))
        pool_acc[...] += lax.dot_general(onehot, z2, dn,
                                         preferred_element_type=jnp.float32)
        cnt_acc[...] += lax.dot_general(onehot, jnp.ones((_BN, _H), jnp.float32),
                                        dn, preferred_element_type=jnp.float32)

        @pl.when(i == nb - 1)
        def _():
            mean = pool_acc[...] / jnp.maximum(cnt_acc[...], 1.0)
            y = lax.dot_general(mean, wp_ref[...], (((1,), (1,)), ((), ())),
                                preferred_element_type=jnp.float32) + bp_ref[...]
            o_ref[...] = jax.nn.sigmoid(y)

    return pl.pallas_call(
        body,
        grid=(nb,),
        in_specs=[
            pl.BlockSpec(memory_space=pltpu.SMEM),
            pl.BlockSpec((_BN, _H), lambda i: (i, 0)),
            pl.BlockSpec((_BN, _H), lambda i: (i, 0)),
            pl.BlockSpec((_BN, _H), lambda i: (i + nb, 0)),
            pl.BlockSpec((_H, _H), lambda i: (0, 0)),
            pl.BlockSpec((_H, _H), lambda i: (0, 0)),
            pl.BlockSpec((1, _H), lambda i: (0, 0)),
            pl.BlockSpec((_BN, 1), lambda i: (i, 0)),
            pl.BlockSpec((_C, _H), lambda i: (0, 0)),
            pl.BlockSpec((1, _C), lambda i: (0, 0)),
        ],
        out_specs=pl.BlockSpec((_G, _C), lambda i: (0, 0)),
        out_shape=jax.ShapeDtypeStruct((_G, _C), jnp.float32),
        scratch_shapes=[
            pltpu.VMEM((_G, _H), jnp.float32),
            pltpu.VMEM((_G, _H), jnp.float32),
        ],
    )(eps.reshape(1, 1), zin, parts, parts, W, mask, b.reshape(1, _H),
      batch_col, Wp, bp.reshape(1, _C))


def kernel(x, edge_index, batch, weights, W1, b1, eps1, W2, b2, eps2, Wp, bp):
    src = edge_index[0]
    dst = edge_index[1]
    batch_col = batch.reshape(_N, 1)
    parts1 = _sc_aggregate(x, src, dst)
    z1 = _tc_layer1(x, parts1, W1, weights, b1, eps1)
    parts2 = _sc_aggregate(z1, src, dst)
    return _tc_layer2_pool(z1, parts2, W2, weights, b2, eps2,
                           batch_col, Wp, bp)


# speedup vs baseline: 14.4614x; 1.0024x over previous
import functools

import jax
import jax.numpy as jnp
from jax import lax
from jax.experimental import pallas as pl
from jax.experimental.pallas import tpu as pltpu
from jax.experimental.pallas import tpu_sc as plsc

_N = 10000
_E = 320000
_H = 128
_G = 128
_C = 10

_NC = 2
_NS = 16
_EPC = _E // _NC
_EPW = _E // (_NC * _NS)
_CHUNK = 104
_RPW = 624
_RTAIL = _N - _NS * _RPW
_NW = _NC * _NS
_EPW2 = 9984
_NCH = _EPW2 // _CHUNK
_NBUF = 3
_XBASE = _NW * _EPW2
_XCH = 64

_BN = 2000


def _sc_aggregate(xin, src, dst):
    mesh = plsc.VectorSubcoreMesh(core_axis_name="c", subcore_axis_name="s")

    @functools.partial(
        pl.kernel,
        out_type=jax.ShapeDtypeStruct((_NC * _N, _H), jnp.float32),
        mesh=mesh,
        scratch_types=[
            pltpu.VMEM((_EPW2,), jnp.int32),
            pltpu.VMEM((_NBUF, _CHUNK), jnp.int32),
            pltpu.VMEM((_NBUF, _CHUNK, _H), jnp.float32),
            pltpu.VMEM((_XCH,), jnp.int32),
            pltpu.VMEM((_XCH,), jnp.int32),
            pltpu.VMEM_SHARED((_N, _H), jnp.float32),
            pltpu.SemaphoreType.DMA,
            pltpu.SemaphoreType.DMA((_NBUF,)),
            pltpu.SemaphoreType.DMA((_NBUF,)),
            pltpu.SemaphoreType.DMA((_NBUF,)),
        ],
    )
    def agg(x_hbm, src_hbm, dst_hbm, out_hbm,
            srcs_v, dring, rows, srcx_v, dstx_v, acc_sh,
            sem_s, gsem, dsem, ssem):
        c = lax.axis_index("c")
        s = lax.axis_index("s")
        w = c * _NS + s
        ebase = w * _EPW2

        def fire_chunk(j, b):
            pltpu.async_copy(dst_hbm.at[pl.ds(ebase + j * _CHUNK, _CHUNK)],
                             dring.at[b], dsem.at[b])
            pltpu.async_copy(x_hbm.at[srcs_v.at[pl.ds(j * _CHUNK, 48)]],
                             rows.at[b].at[pl.ds(0, 48)], gsem.at[b])
            pltpu.async_copy(x_hbm.at[srcs_v.at[pl.ds(j * _CHUNK + 48, 56)]],
                             rows.at[b].at[pl.ds(48, 56)], gsem.at[b])

        def wait_gather(j, b):
            pltpu.make_async_copy(
                x_hbm.at[srcs_v.at[pl.ds(j * _CHUNK, _CHUNK)]],
                rows.at[b], gsem.at[b]).wait()
            pltpu.make_async_copy(
                dst_hbm.at[pl.ds(ebase + j * _CHUNK, _CHUNK)],
                dring.at[b], dsem.at[b]).wait()

        def wait_scatter(b):
            pltpu.make_async_copy(rows.at[b], acc_sh.at[dring.at[b]],
                                  ssem.at[b]).wait()


        stage = pltpu.async_copy(src_hbm.at[pl.ds(ebase, _EPW2)], srcs_v, sem_s)

        def zero_body(i, _):
            for j in range(_H // 16):
                rows[2, i, pl.ds(j * 16, 16)] = jnp.zeros((16,), jnp.float32)
            return 0
        lax.fori_loop(0, _CHUNK, zero_body, 0)
        stage.wait()
        for b in range(_NBUF - 1):
            fire_chunk(b, b)
        r0 = s * _RPW
        for k in range(_RPW // _CHUNK):
            pltpu.async_copy(rows.at[2].at[pl.ds(0, _CHUNK)],
                            acc_sh.at[pl.ds(r0 + k * _CHUNK, _CHUNK)], ssem.at[2])

        @pl.when(s == _NS - 1)
        def _():
            pltpu.async_copy(rows.at[2].at[pl.ds(0, _RTAIL)],
                            acc_sh.at[pl.ds(_NS * _RPW, _RTAIL)], ssem.at[2])
        for k in range(_RPW // _CHUNK):
            pltpu.make_async_copy(rows.at[2].at[pl.ds(0, _CHUNK)],
                                  acc_sh.at[pl.ds(r0 + k * _CHUNK, _CHUNK)],
                                  ssem.at[2]).wait()

        @pl.when(s == _NS - 1)
        def _():
            pltpu.make_async_copy(rows.at[2].at[pl.ds(0, _RTAIL)],
                                  acc_sh.at[pl.ds(_NS * _RPW, _RTAIL)],
                                  ssem.at[2]).wait()
        plsc.subcore_barrier()

        def body(t, _):
            for b in range(_NBUF):
                j = t * _NBUF + b
                wait_gather(j, b)
                pltpu.async_copy(rows.at[b], acc_sh.at[dring.at[b]],
                                 ssem.at[b], add=True)
                b2 = (b + _NBUF - 1) % _NBUF

                @pl.when(j >= 1)
                def _():
                    wait_scatter(b2)

                @pl.when(j + 2 < _NCH)
                def _():
                    fire_chunk(j + 2, b2)
            return 0
        lax.fori_loop(0, _NCH // _NBUF, body, 0)

        wait_scatter((_NCH - 1) % _NBUF)

        @pl.when(w < 8)
        def _():
            e0 = _XBASE + w * _XCH
            pltpu.sync_copy(src_hbm.at[pl.ds(e0, _XCH)], srcx_v)
            pltpu.sync_copy(dst_hbm.at[pl.ds(e0, _XCH)], dstx_v)
            pltpu.async_copy(x_hbm.at[srcx_v],
                             rows.at[0].at[pl.ds(0, _XCH)], gsem.at[0]).wait()
            pltpu.sync_copy(rows.at[0].at[pl.ds(0, _XCH)],
                            acc_sh.at[dstx_v], add=True)

        plsc.subcore_barrier()
        pltpu.sync_copy(acc_sh.at[pl.ds(r0, _RPW)],
                        out_hbm.at[pl.ds(c * _N + r0, _RPW)])

        @pl.when(s == _NS - 1)
        def _():
            pltpu.sync_copy(acc_sh.at[pl.ds(_NS * _RPW, _RTAIL)],
                            out_hbm.at[pl.ds(c * _N + _NS * _RPW, _RTAIL)])

    return agg(xin, src, dst)


def _gin_dense(z, agg0, agg1, wm, bias_row, eps_ref):
    return jnp.maximum(
        lax.dot_general((1.0 + eps_ref[0, 0]) * z + agg0 + agg1, wm,
                        (((1,), (1,)), ((), ())),
                        preferred_element_type=jnp.float32) + bias_row, 0.0)


def _tc_layer1(xin, parts, W, mask, b, eps):
    def body(eps_ref, x_ref, p0_ref, p1_ref, w_ref, m_ref, b_ref, o_ref):
        wm = w_ref[...] * m_ref[...]
        o_ref[...] = _gin_dense(x_ref[...], p0_ref[...], p1_ref[...], wm,
                                b_ref[...], eps_ref)

    nb = _N // _BN
    return pl.pallas_call(
        body,
        grid=(nb,),
        in_specs=[
            pl.BlockSpec(memory_space=pltpu.SMEM),
            pl.BlockSpec((_BN, _H), lambda i: (i, 0)),
            pl.BlockSpec((_BN, _H), lambda i: (i, 0)),
            pl.BlockSpec((_BN, _H), lambda i: (i + nb, 0)),
            pl.BlockSpec((_H, _H), lambda i: (0, 0)),
            pl.BlockSpec((_H, _H), lambda i: (0, 0)),
            pl.BlockSpec((1, _H), lambda i: (0, 0)),
        ],
        out_specs=pl.BlockSpec((_BN, _H), lambda i: (i, 0)),
        out_shape=jax.ShapeDtypeStruct((_N, _H), jnp.float32),
    )(eps.reshape(1, 1), xin, parts, parts, W, mask, b.reshape(1, _H))


def _tc_layer2_pool(zin, parts, W, mask, b, eps, batch_col, Wp, bp):
    nb = _N // _BN

    def body(eps_ref, z_ref, p0_ref, p1_ref, w_ref, m_ref, b_ref,
             bt_ref, wp_ref, bp_ref, o_ref, pool_acc, cnt_acc):
        i = pl.program_id(0)
        wm = w_ref[...] * m_ref[...]
        z2 = _gin_dense(z_ref[...], p0_ref[...], p1_ref[...], wm,
                        b_ref[...], eps_ref)
        gid = lax.broadcasted_iota(jnp.int32, (_BN, _G), 1)
        onehot = (bt_ref[...] == gid).astype(jnp.float32)

        @pl.when(i == 0)
        def _():
            pool_acc[...] = jnp.zeros_like(pool_acc)
            cnt_acc[...] = jnp.zeros_like(cnt_acc)

        dn = (((0,), (0,)), ((), ()))
        pool_acc[...] += lax.dot_general(onehot, z2, dn,
                                         preferred_element_type=jnp.float32)
        cnt_acc[...] += lax.dot_general(onehot, jnp.ones((_BN, _H), jnp.float32),
                                        dn, preferred_element_type=jnp.float32)

        @pl.when(i == nb - 1)
        def _():
            mean = pool_acc[...] / jnp.maximum(cnt_acc[...], 1.0)
            y = lax.dot_general(mean, wp_ref[...], (((1,), (1,)), ((), ())),
                                preferred_element_type=jnp.float32) + bp_ref[...]
            o_ref[...] = jax.nn.sigmoid(y)

    return pl.pallas_call(
        body,
        grid=(nb,),
        in_specs=[
            pl.BlockSpec(memory_space=pltpu.SMEM),
            pl.BlockSpec((_BN, _H), lambda i: (i, 0)),
            pl.BlockSpec((_BN, _H), lambda i: (i, 0)),
            pl.BlockSpec((_BN, _H), lambda i: (i + nb, 0)),
            pl.BlockSpec((_H, _H), lambda i: (0, 0)),
            pl.BlockSpec((_H, _H), lambda i: (0, 0)),
            pl.BlockSpec((1, _H), lambda i: (0, 0)),
            pl.BlockSpec((_BN, 1), lambda i: (i, 0)),
            pl.BlockSpec((_C, _H), lambda i: (0, 0)),
            pl.BlockSpec((1, _C), lambda i: (0, 0)),
        ],
        out_specs=pl.BlockSpec((_G, _C), lambda i: (0, 0)),
        out_shape=jax.ShapeDtypeStruct((_G, _C), jnp.float32),
        scratch_shapes=[
            pltpu.VMEM((_G, _H), jnp.float32),
            pltpu.VMEM((_G, _H), jnp.float32),
        ],
    )(eps.reshape(1, 1), zin, parts, parts, W, mask, b.reshape(1, _H),
      batch_col, Wp, bp.reshape(1, _C))


def kernel(x, edge_index, batch, weights, W1, b1, eps1, W2, b2, eps2, Wp, bp):
    src = edge_index[0]
    dst = edge_index[1]
    batch_col = batch.reshape(_N, 1)
    parts1 = _sc_aggregate(x, src, dst)
    z1 = _tc_layer1(x, parts1, W1, weights, b1, eps1)
    parts2 = _sc_aggregate(z1, src, dst)
    return _tc_layer2_pool(z1, parts2, W2, weights, b2, eps2,
                           batch_col, Wp, bp)


# speedup vs baseline: 15.0592x; 1.0413x over previous
import functools

import jax
import jax.numpy as jnp
from jax import lax
from jax.experimental import pallas as pl
from jax.experimental.pallas import tpu as pltpu
from jax.experimental.pallas import tpu_sc as plsc

_N = 10000
_E = 320000
_H = 128
_G = 128
_C = 10

_NC = 2
_NS = 16
_EPC = _E // _NC
_EPW = _E // (_NC * _NS)
_CHUNK = 104
_RPW = 624
_RTAIL = _N - _NS * _RPW
_NW = _NC * _NS
_EPW2 = 9984
_NCH = _EPW2 // _CHUNK
_NBUF = 3
_XBASE = _NW * _EPW2
_XCH = 64

_BN = 2000


def _sc_aggregate(xin, edge_flat):
    mesh = plsc.VectorSubcoreMesh(core_axis_name="c", subcore_axis_name="s")

    @functools.partial(
        pl.kernel,
        out_type=jax.ShapeDtypeStruct((_NC * _N, _H), jnp.float32),
        mesh=mesh,
        scratch_types=[
            pltpu.VMEM((_EPW2,), jnp.int32),
            pltpu.VMEM((_NBUF, _CHUNK), jnp.int32),
            pltpu.VMEM((_NBUF, _CHUNK, _H), jnp.float32),
            pltpu.VMEM((_XCH,), jnp.int32),
            pltpu.VMEM((_XCH,), jnp.int32),
            pltpu.VMEM_SHARED((_N, _H), jnp.float32),
            pltpu.SemaphoreType.DMA,
            pltpu.SemaphoreType.DMA((_NBUF,)),
            pltpu.SemaphoreType.DMA((_NBUF,)),
            pltpu.SemaphoreType.DMA((_NBUF,)),
        ],
    )
    def agg(x_hbm, edge_hbm, out_hbm,
            srcs_v, dring, rows, srcx_v, dstx_v, acc_sh,
            sem_s, gsem, dsem, ssem):
        c = lax.axis_index("c")
        s = lax.axis_index("s")
        w = c * _NS + s
        ebase = w * _EPW2

        def fire_chunk(j, b):
            pltpu.async_copy(edge_hbm.at[pl.ds(_E + ebase + j * _CHUNK, _CHUNK)],
                             dring.at[b], dsem.at[b])
            pltpu.async_copy(x_hbm.at[srcs_v.at[pl.ds(j * _CHUNK, 48)]],
                             rows.at[b].at[pl.ds(0, 48)], gsem.at[b])
            pltpu.async_copy(x_hbm.at[srcs_v.at[pl.ds(j * _CHUNK + 48, 56)]],
                             rows.at[b].at[pl.ds(48, 56)], gsem.at[b])

        def wait_gather(j, b):
            pltpu.make_async_copy(
                x_hbm.at[srcs_v.at[pl.ds(j * _CHUNK, _CHUNK)]],
                rows.at[b], gsem.at[b]).wait()
            pltpu.make_async_copy(
                edge_hbm.at[pl.ds(_E + ebase + j * _CHUNK, _CHUNK)],
                dring.at[b], dsem.at[b]).wait()

        def wait_scatter(b):
            pltpu.make_async_copy(rows.at[b], acc_sh.at[dring.at[b]],
                                  ssem.at[b]).wait()


        stage = pltpu.async_copy(edge_hbm.at[pl.ds(ebase, _EPW2)], srcs_v, sem_s)

        def zero_body(i, _):
            for j in range(_H // 16):
                rows[2, i, pl.ds(j * 16, 16)] = jnp.zeros((16,), jnp.float32)
            return 0
        lax.fori_loop(0, _CHUNK, zero_body, 0)
        stage.wait()
        for b in range(_NBUF - 1):
            fire_chunk(b, b)
        r0 = s * _RPW
        for k in range(_RPW // _CHUNK):
            pltpu.async_copy(rows.at[2].at[pl.ds(0, _CHUNK)],
                            acc_sh.at[pl.ds(r0 + k * _CHUNK, _CHUNK)], ssem.at[2])

        @pl.when(s == _NS - 1)
        def _():
            pltpu.async_copy(rows.at[2].at[pl.ds(0, _RTAIL)],
                            acc_sh.at[pl.ds(_NS * _RPW, _RTAIL)], ssem.at[2])
        for k in range(_RPW // _CHUNK):
            pltpu.make_async_copy(rows.at[2].at[pl.ds(0, _CHUNK)],
                                  acc_sh.at[pl.ds(r0 + k * _CHUNK, _CHUNK)],
                                  ssem.at[2]).wait()

        @pl.when(s == _NS - 1)
        def _():
            pltpu.make_async_copy(rows.at[2].at[pl.ds(0, _RTAIL)],
                                  acc_sh.at[pl.ds(_NS * _RPW, _RTAIL)],
                                  ssem.at[2]).wait()
        plsc.subcore_barrier()

        def body(t, _):
            for b in range(_NBUF):
                j = t * _NBUF + b
                wait_gather(j, b)
                pltpu.async_copy(rows.at[b], acc_sh.at[dring.at[b]],
                                 ssem.at[b], add=True)
                b2 = (b + _NBUF - 1) % _NBUF

                @pl.when(j >= 1)
                def _():
                    wait_scatter(b2)

                @pl.when(j + 2 < _NCH)
                def _():
                    fire_chunk(j + 2, b2)
            return 0
        lax.fori_loop(0, _NCH // _NBUF, body, 0)

        wait_scatter((_NCH - 1) % _NBUF)

        @pl.when(w < 8)
        def _():
            e0 = _XBASE + w * _XCH
            pltpu.sync_copy(edge_hbm.at[pl.ds(e0, _XCH)], srcx_v)
            pltpu.sync_copy(edge_hbm.at[pl.ds(_E + e0, _XCH)], dstx_v)
            pltpu.async_copy(x_hbm.at[srcx_v],
                             rows.at[0].at[pl.ds(0, _XCH)], gsem.at[0]).wait()
            pltpu.sync_copy(rows.at[0].at[pl.ds(0, _XCH)],
                            acc_sh.at[dstx_v], add=True)

        plsc.subcore_barrier()
        pltpu.sync_copy(acc_sh.at[pl.ds(r0, _RPW)],
                        out_hbm.at[pl.ds(c * _N + r0, _RPW)])

        @pl.when(s == _NS - 1)
        def _():
            pltpu.sync_copy(acc_sh.at[pl.ds(_NS * _RPW, _RTAIL)],
                            out_hbm.at[pl.ds(c * _N + _NS * _RPW, _RTAIL)])

    return agg(xin, edge_flat)


def _gin_dense(z, agg0, agg1, wm, bias_row, eps_ref):
    return jnp.maximum(
        lax.dot_general((1.0 + eps_ref[0, 0]) * z + agg0 + agg1, wm,
                        (((1,), (1,)), ((), ())),
                        preferred_element_type=jnp.float32) + bias_row, 0.0)


def _tc_layer1(xin, parts, W, mask, b, eps):
    def body(eps_ref, x_ref, p0_ref, p1_ref, w_ref, m_ref, b_ref, o_ref):
        wm = w_ref[...] * m_ref[...]
        o_ref[...] = _gin_dense(x_ref[...], p0_ref[...], p1_ref[...], wm,
                                b_ref[...], eps_ref)

    nb = _N // _BN
    return pl.pallas_call(
        body,
        grid=(nb,),
        in_specs=[
            pl.BlockSpec(memory_space=pltpu.SMEM),
            pl.BlockSpec((_BN, _H), lambda i: (i, 0)),
            pl.BlockSpec((_BN, _H), lambda i: (i, 0)),
            pl.BlockSpec((_BN, _H), lambda i: (i + nb, 0)),
            pl.BlockSpec((_H, _H), lambda i: (0, 0)),
            pl.BlockSpec((_H, _H), lambda i: (0, 0)),
            pl.BlockSpec((1, _H), lambda i: (0, 0)),
        ],
        out_specs=pl.BlockSpec((_BN, _H), lambda i: (i, 0)),
        out_shape=jax.ShapeDtypeStruct((_N, _H), jnp.float32),
    )(eps.reshape(1, 1), xin, parts, parts, W, mask, b.reshape(1, _H))


def _tc_layer2_pool(zin, parts, W, mask, b, eps, batch_col, Wp, bp):
    nb = _N // _BN

    def body(eps_ref, z_ref, p0_ref, p1_ref, w_ref, m_ref, b_ref,
             bt_ref, wp_ref, bp_ref, o_ref, pool_acc, cnt_acc):
        i = pl.program_id(0)
        wm = w_ref[...] * m_ref[...]
        z2 = _gin_dense(z_ref[...], p0_ref[...], p1_ref[...], wm,
                        b_ref[...], eps_ref)
        gid = lax.broadcasted_iota(jnp.int32, (_BN, _G), 1)
        onehot = (bt_ref[...] == gid).astype(jnp.float32)

        @pl.when(i == 0)
        def _():
            pool_acc[...] = jnp.zeros_like(pool_acc)
            cnt_acc[...] = jnp.zeros_like(cnt_acc)

        dn = (((0,), (0,)), ((), ()))
        pool_acc[...] += lax.dot_general(onehot, z2, dn,
                                         preferred_element_type=jnp.float32)
        cnt_acc[...] += lax.dot_general(onehot, jnp.ones((_BN, _H), jnp.float32),
                                        dn, preferred_element_type=jnp.float32)

        @pl.when(i == nb - 1)
        def _():
            mean = pool_acc[...] / jnp.maximum(cnt_acc[...], 1.0)
            y = lax.dot_general(mean, wp_ref[...], (((1,), (1,)), ((), ())),
                                preferred_element_type=jnp.float32) + bp_ref[...]
            o_ref[...] = jax.nn.sigmoid(y)

    return pl.pallas_call(
        body,
        grid=(nb,),
        in_specs=[
            pl.BlockSpec(memory_space=pltpu.SMEM),
            pl.BlockSpec((_BN, _H), lambda i: (i, 0)),
            pl.BlockSpec((_BN, _H), lambda i: (i, 0)),
            pl.BlockSpec((_BN, _H), lambda i: (i + nb, 0)),
            pl.BlockSpec((_H, _H), lambda i: (0, 0)),
            pl.BlockSpec((_H, _H), lambda i: (0, 0)),
            pl.BlockSpec((1, _H), lambda i: (0, 0)),
            pl.BlockSpec((_BN, 1), lambda i: (i, 0)),
            pl.BlockSpec((_C, _H), lambda i: (0, 0)),
            pl.BlockSpec((1, _C), lambda i: (0, 0)),
        ],
        out_specs=pl.BlockSpec((_G, _C), lambda i: (0, 0)),
        out_shape=jax.ShapeDtypeStruct((_G, _C), jnp.float32),
        scratch_shapes=[
            pltpu.VMEM((_G, _H), jnp.float32),
            pltpu.VMEM((_G, _H), jnp.float32),
        ],
    )(eps.reshape(1, 1), zin, parts, parts, W, mask, b.reshape(1, _H),
      batch_col, Wp, bp.reshape(1, _C))


def kernel(x, edge_index, batch, weights, W1, b1, eps1, W2, b2, eps2, Wp, bp):
    edge_flat = edge_index.reshape(2 * _E)
    batch_col = batch.reshape(_N, 1)
    parts1 = _sc_aggregate(x, edge_flat)
    z1 = _tc_layer1(x, parts1, W1, weights, b1, eps1)
    parts2 = _sc_aggregate(z1, edge_flat)
    return _tc_layer2_pool(z1, parts2, W2, weights, b2, eps2,
                           batch_col, Wp, bp)
